# baseline (speedup 1.0000x reference)
import jax
import jax.numpy as jnp
from jax.experimental import pallas as pl
from jax.experimental.pallas import tpu as pltpu

R, N, D, B, L, S = 3, 1024, 256, 2, 2, 512

_INTERPRET = False


def _dot(x, w):
    return jax.lax.dot_general(x, w, (((1,), (0,)), ((), ())),
                               preferred_element_type=jnp.float32)


def _dot_t(x, w):
    return jax.lax.dot_general(x, w, (((1,), (1,)), ((), ())),
                               preferred_element_type=jnp.float32)


def _sum11(x):
    return jnp.sum(x, axis=1, keepdims=True).sum(axis=0, keepdims=True)


def _fused_kernel(rc_ref, hc_ref, rb_ref, hb_ref, s1_ref, s2_ref, a_ref, a2_ref,
                  w1_ref, w2_ref, f1w_ref, f1b_ref, f2w_ref, f2b_ref,
                  fkw_ref, fkb_ref,
                  logits_ref, misc_ref,
                  shp1, shp2, shn1, shn2, scp1, scp2, ssq):
    r = pl.program_id(0)
    rc = rc_ref[0]
    hc = hc_ref[0]

    def wmat(c, b_ref, l):
        return c[l:l + 1, 0:1] * b_ref[l, 0] + c[l:l + 1, 1:2] * b_ref[l, 1]

    a = a_ref[0]
    a2 = a2_ref[0]
    x1 = s1_ref[0]
    x2 = s2_ref[0]

    def layer(adj, u, v, w):
        y = _dot(adj, jnp.concatenate([_dot(u, w), _dot(v, w)], axis=1))
        y = jnp.maximum(y, 0.0)
        return y[:, :D], y[:, D:]

    p1, q1 = layer(a, x1, x2, wmat(rc, rb_ref, 0))
    p2, q2 = layer(a2, x1, x2, wmat(hc, hb_ref, 0))
    p1, q1 = layer(a, p1, q1, wmat(rc, rb_ref, 1))
    p2, q2 = layer(a2, p2, q2, wmat(hc, hb_ref, 1))

    cp1 = jax.nn.sigmoid(jnp.mean(p1, axis=0, keepdims=True))
    cp2 = jax.nn.sigmoid(jnp.mean(p2, axis=0, keepdims=True))

    w1 = w1_ref[...]
    w2 = w2_ref[...]

    def seg(c, h, w):
        return _dot_t(_dot_t(c, w), h)

    def disc_block(c1, c2, hb1, hb2, he1, he2):
        return jnp.concatenate(
            [seg(c1, hb2, w1), seg(c2, hb1, w2),
             seg(c1, he2, w1), seg(c2, he1, w2)], axis=0)[None]

    logits_ref[pl.ds(1 + r, 1)] = disc_block(cp1, cp2, p1, p2, q1, q2)

    hp_i = (p1 + p2) * 0.5
    hn_i = (q1 + q2) * 0.5
    sq_r = _sum11(hp_i * hp_i - hn_i * hn_i)

    @pl.when(r == 0)
    def _():
        shp1[...] = p1
        shp2[...] = p2
        shn1[...] = q1
        shn2[...] = q2
        scp1[...] = cp1
        scp2[...] = cp2
        ssq[...] = sq_r

    @pl.when(r > 0)
    def _():
        shp1[...] += p1
        shp2[...] += p2
        shn1[...] += q1
        shn2[...] += q2
        scp1[...] += cp1
        scp2[...] += cp2
        ssq[...] += sq_r

    @pl.when(r == R - 1)
    def _():
        inv = 1.0 / R
        hp1_all = shp1[...] * inv
        hp2_all = shp2[...] * inv
        hn1_all = shn1[...] * inv
        hn2_all = shn2[...] * inv
        c1_all = scp1[...] * inv
        c2_all = scp2[...] * inv

        logits_ref[pl.ds(0, 1)] = disc_block(c1_all, c2_all, hp1_all, hp2_all,
                                             hn1_all, hn2_all)

        amat = (hp1_all + hp2_all) * 0.5
        aneg = (hn1_all + hn2_all) * 0.5
        reg = ssq[...] - 2.0 * R * _sum11(amat * (amat - aneg))

        def proj(h):
            z = _dot_t(h, f1w_ref[...]) + f1b_ref[...]
            z = jnp.where(z > 0.0, z, jnp.exp(jnp.minimum(z, 0.0)) - 1.0)
            return _dot_t(z, f2w_ref[...]) + f2b_ref[...]

        z1 = proj(hp1_all)
        z2 = proj(hp2_all)
        zk = _dot(z1, fkw_ref[...])
        fkb = fkb_ref[0:1, 0:1]

        m1 = _dot_t(zk, z1)
        m2 = _dot_t(zk, z2)

        rowi = jax.lax.broadcasted_iota(jnp.int32, (N, N), 0)
        coli = jax.lax.broadcasted_iota(jnp.int32, (N, N), 1)
        mask7 = coli == ((7 * rowi + 1) & (N - 1))
        mask13 = coli == ((13 * rowi + 5) & (N - 1))

        def diag(m, mask):
            return jnp.sum(jnp.where(mask, m, 0.0), axis=1, keepdims=True) + fkb

        d7_1 = diag(m1, mask7)
        d7_2 = diag(m2, mask7)
        d13_1 = diag(m1, mask13)
        d13_2 = diag(m2, mask13)

        def bce_pos(v):
            return _sum11(jnp.maximum(v, 0.0) - v
                          + jnp.log1p(jnp.exp(-jnp.abs(v))))

        def bce_neg(v):
            return _sum11(jnp.maximum(v, 0.0) + jnp.log1p(jnp.exp(-jnp.abs(v))))

        node = jnp.zeros((1, 1), jnp.float32)
        for i in range(R):
            a0 = 17 * i
            node = node + (bce_pos(d7_1[a0:a0 + S]) + bce_pos(d7_2[a0:a0 + S])
                           + bce_neg(d13_1[a0:a0 + S])
                           + bce_neg(d13_2[a0:a0 + S])) * (1.0 / (4 * S))

        lane = jax.lax.broadcasted_iota(jnp.int32, (8, 128), 1)
        sub = jax.lax.broadcasted_iota(jnp.int32, (8, 128), 0)
        regb = jnp.broadcast_to(reg, (8, 128))
        nodeb = jnp.broadcast_to(node, (8, 128))
        misc_ref[...] = jnp.where((sub == 0) & (lane == 0), regb,
                                  jnp.where((sub == 0) & (lane == 1), nodeb,
                                            0.0))


def kernel(seq1, seq2, adj, adj_2, sparse, rgcn_bases, rgcn_comp, hrgcn_bases,
           hrgcn_comp, disc_W1, disc_W2, fc1_w, fc1_b, fc2_w, fc2_b, fk_w, fk_b):
    rc_t = jnp.transpose(rgcn_comp, (1, 0, 2))
    hc_t = jnp.transpose(hrgcn_comp, (1, 0, 2))

    full = lambda shape: pl.BlockSpec(shape, lambda r: (0,) * len(shape))
    per_r3 = lambda d1, d2: pl.BlockSpec((1, d1, d2), lambda r: (r, 0, 0))

    logits, misc = pl.pallas_call(
        _fused_kernel,
        grid=(R,),
        in_specs=[
            per_r3(L, B), per_r3(L, B),
            full((L, B, D, D)), full((L, B, D, D)),
            per_r3(N, D), per_r3(N, D),
            per_r3(N, N), per_r3(N, N),
            full((D, D)), full((D, D)),
            full((D, D)), full((1, D)), full((D, D)), full((1, D)),
            full((D, D)), full((1, 1)),
        ],
        out_specs=[full((4, 4, N)), full((8, 128))],
        out_shape=[jax.ShapeDtypeStruct((4, 4, N), jnp.float32),
                   jax.ShapeDtypeStruct((8, 128), jnp.float32)],
        scratch_shapes=[pltpu.VMEM((N, D), jnp.float32)] * 4
        + [pltpu.VMEM((1, D), jnp.float32)] * 2
        + [pltpu.VMEM((1, 1), jnp.float32)],
        interpret=_INTERPRET,
    )(rc_t, hc_t, rgcn_bases, hrgcn_bases, seq1, seq2, adj, adj_2,
      disc_W1, disc_W2, fc1_w, fc1_b.reshape(1, D), fc2_w, fc2_b.reshape(1, D),
      fk_w, fk_b.reshape(1, 1))

    return jnp.concatenate([logits.reshape(-1), misc[0, :2]])


# speedup vs baseline: 1.0045x; 1.0045x over previous
import jax
import jax.numpy as jnp
from jax.experimental import pallas as pl
from jax.experimental.pallas import tpu as pltpu

R, N, D, B, L, S = 3, 1024, 256, 2, 2, 512

_INTERPRET = False


def _dot(x, w):
    return jax.lax.dot_general(x, w, (((1,), (0,)), ((), ())),
                               preferred_element_type=jnp.float32)


def _dot_t(x, w):
    return jax.lax.dot_general(x, w, (((1,), (1,)), ((), ())),
                               preferred_element_type=jnp.float32)


def _sum11(x):
    return jnp.sum(x, axis=1, keepdims=True).sum(axis=0, keepdims=True)


def _fused_kernel(rc_ref, hc_ref, rb_ref, hb_ref, s1_ref, s2_ref, a_ref, a2_ref,
                  w1_ref, w2_ref, f1w_ref, f1b_ref, f2w_ref, f2b_ref,
                  fkw_ref, fkb_ref,
                  logits_ref, misc_ref,
                  shp1, shp2, shn1, shn2, scp1, scp2, ssq):
    r = pl.program_id(0)
    rc = rc_ref[0]
    hc = hc_ref[0]

    def wmat(c, b_ref, l):
        return c[l:l + 1, 0:1] * b_ref[l, 0] + c[l:l + 1, 1:2] * b_ref[l, 1]

    a = a_ref[0]
    a2 = a2_ref[0]
    x1 = s1_ref[0]
    x2 = s2_ref[0]

    def layer(adj, u, v, w):
        y = _dot(adj, jnp.concatenate([_dot(u, w), _dot(v, w)], axis=1))
        y = jnp.maximum(y, 0.0)
        return y[:, :D], y[:, D:]

    p1, q1 = layer(a, x1, x2, wmat(rc, rb_ref, 0))
    p2, q2 = layer(a2, x1, x2, wmat(hc, hb_ref, 0))
    p1, q1 = layer(a, p1, q1, wmat(rc, rb_ref, 1))
    p2, q2 = layer(a2, p2, q2, wmat(hc, hb_ref, 1))

    cp1 = jax.nn.sigmoid(jnp.mean(p1, axis=0, keepdims=True))
    cp2 = jax.nn.sigmoid(jnp.mean(p2, axis=0, keepdims=True))

    w1 = w1_ref[...]
    w2 = w2_ref[...]

    def seg(c, h, w):
        return _dot_t(_dot_t(c, w), h)

    def disc_block(c1, c2, hb1, hb2, he1, he2):
        return jnp.concatenate(
            [seg(c1, hb2, w1), seg(c2, hb1, w2),
             seg(c1, he2, w1), seg(c2, he1, w2)], axis=0)[None]

    logits_ref[pl.ds(1 + r, 1)] = disc_block(cp1, cp2, p1, p2, q1, q2)

    hp_i = (p1 + p2) * 0.5
    hn_i = (q1 + q2) * 0.5
    sq_r = _sum11(hp_i * hp_i - hn_i * hn_i)

    @pl.when(r == 0)
    def _():
        shp1[...] = p1
        shp2[...] = p2
        shn1[...] = q1
        shn2[...] = q2
        scp1[...] = cp1
        scp2[...] = cp2
        ssq[...] = sq_r

    @pl.when(r > 0)
    def _():
        shp1[...] += p1
        shp2[...] += p2
        shn1[...] += q1
        shn2[...] += q2
        scp1[...] += cp1
        scp2[...] += cp2
        ssq[...] += sq_r

    @pl.when(r == R - 1)
    def _():
        inv = 1.0 / R
        hp1_all = shp1[...] * inv
        hp2_all = shp2[...] * inv
        hn1_all = shn1[...] * inv
        hn2_all = shn2[...] * inv
        c1_all = scp1[...] * inv
        c2_all = scp2[...] * inv

        logits_ref[pl.ds(0, 1)] = disc_block(c1_all, c2_all, hp1_all, hp2_all,
                                             hn1_all, hn2_all)

        amat = (hp1_all + hp2_all) * 0.5
        aneg = (hn1_all + hn2_all) * 0.5
        reg = ssq[...] - 2.0 * R * _sum11(amat * (amat - aneg))

        def proj(h):
            z = _dot_t(h, f1w_ref[...]) + f1b_ref[...]
            z = jnp.where(z > 0.0, z, jnp.exp(jnp.minimum(z, 0.0)) - 1.0)
            return _dot_t(z, f2w_ref[...]) + f2b_ref[...]

        z1 = proj(hp1_all)
        z2 = proj(hp2_all)
        zk = _dot(z1, fkw_ref[...])
        fkb = fkb_ref[0:1, 0:1]

        NR = 576
        zks = zk[:NR]
        m1 = _dot_t(zks, z1)
        m2 = _dot_t(zks, z2)

        rowi = jax.lax.broadcasted_iota(jnp.int32, (NR, N), 0)
        coli = jax.lax.broadcasted_iota(jnp.int32, (NR, N), 1)
        mask7 = coli == ((7 * rowi + 1) & (N - 1))
        mask13 = coli == ((13 * rowi + 5) & (N - 1))

        def diag(m, mask):
            return jnp.sum(jnp.where(mask, m, 0.0), axis=1, keepdims=True) + fkb

        d7_1 = diag(m1, mask7)
        d7_2 = diag(m2, mask7)
        d13_1 = diag(m1, mask13)
        d13_2 = diag(m2, mask13)

        def bce_pos(v):
            return _sum11(jnp.maximum(v, 0.0) - v
                          + jnp.log1p(jnp.exp(-jnp.abs(v))))

        def bce_neg(v):
            return _sum11(jnp.maximum(v, 0.0) + jnp.log1p(jnp.exp(-jnp.abs(v))))

        node = jnp.zeros((1, 1), jnp.float32)
        for i in range(R):
            a0 = 17 * i
            node = node + (bce_pos(d7_1[a0:a0 + S]) + bce_pos(d7_2[a0:a0 + S])
                           + bce_neg(d13_1[a0:a0 + S])
                           + bce_neg(d13_2[a0:a0 + S])) * (1.0 / (4 * S))

        lane = jax.lax.broadcasted_iota(jnp.int32, (8, 128), 1)
        sub = jax.lax.broadcasted_iota(jnp.int32, (8, 128), 0)
        regb = jnp.broadcast_to(reg, (8, 128))
        nodeb = jnp.broadcast_to(node, (8, 128))
        misc_ref[...] = jnp.where((sub == 0) & (lane == 0), regb,
                                  jnp.where((sub == 0) & (lane == 1), nodeb,
                                            0.0))


def kernel(seq1, seq2, adj, adj_2, sparse, rgcn_bases, rgcn_comp, hrgcn_bases,
           hrgcn_comp, disc_W1, disc_W2, fc1_w, fc1_b, fc2_w, fc2_b, fk_w, fk_b):
    rc_t = jnp.transpose(rgcn_comp, (1, 0, 2))
    hc_t = jnp.transpose(hrgcn_comp, (1, 0, 2))

    full = lambda shape: pl.BlockSpec(shape, lambda r: (0,) * len(shape))
    per_r3 = lambda d1, d2: pl.BlockSpec((1, d1, d2), lambda r: (r, 0, 0))

    logits, misc = pl.pallas_call(
        _fused_kernel,
        grid=(R,),
        in_specs=[
            per_r3(L, B), per_r3(L, B),
            full((L, B, D, D)), full((L, B, D, D)),
            per_r3(N, D), per_r3(N, D),
            per_r3(N, N), per_r3(N, N),
            full((D, D)), full((D, D)),
            full((D, D)), full((1, D)), full((D, D)), full((1, D)),
            full((D, D)), full((1, 1)),
        ],
        out_specs=[full((4, 4, N)), full((8, 128))],
        out_shape=[jax.ShapeDtypeStruct((4, 4, N), jnp.float32),
                   jax.ShapeDtypeStruct((8, 128), jnp.float32)],
        scratch_shapes=[pltpu.VMEM((N, D), jnp.float32)] * 4
        + [pltpu.VMEM((1, D), jnp.float32)] * 2
        + [pltpu.VMEM((1, 1), jnp.float32)],
        interpret=_INTERPRET,
    )(rc_t, hc_t, rgcn_bases, hrgcn_bases, seq1, seq2, adj, adj_2,
      disc_W1, disc_W2, fc1_w, fc1_b.reshape(1, D), fc2_w, fc2_b.reshape(1, D),
      fk_w, fk_b.reshape(1, 1))

    return jnp.concatenate([logits.reshape(-1), misc[0, :2]])


# speedup vs baseline: 1.0106x; 1.0061x over previous
import jax
import jax.numpy as jnp
from jax.experimental import pallas as pl
from jax.experimental.pallas import tpu as pltpu

R, N, D, B, L, S = 3, 1024, 256, 2, 2, 512

_INTERPRET = False


def _dot(x, w):
    return jax.lax.dot_general(x, w, (((1,), (0,)), ((), ())),
                               preferred_element_type=jnp.float32)


def _dotb(x, w):
    return jax.lax.dot_general(x.astype(jnp.bfloat16), w.astype(jnp.bfloat16),
                               (((1,), (0,)), ((), ())),
                               preferred_element_type=jnp.float32)


def _dot_tb(x, w):
    return jax.lax.dot_general(x.astype(jnp.bfloat16), w.astype(jnp.bfloat16),
                               (((1,), (1,)), ((), ())),
                               preferred_element_type=jnp.float32)


def _dot_t(x, w):
    return jax.lax.dot_general(x, w, (((1,), (1,)), ((), ())),
                               preferred_element_type=jnp.float32)


def _sum11(x):
    return jnp.sum(x, axis=1, keepdims=True).sum(axis=0, keepdims=True)


def _fused_kernel(rc_ref, hc_ref, rb_ref, hb_ref, s1_ref, s2_ref, a_ref, a2_ref,
                  w1_ref, w2_ref, f1w_ref, f1b_ref, f2w_ref, f2b_ref,
                  fkw_ref, fkb_ref,
                  logits_ref, misc_ref,
                  shp1, shp2, shn1, shn2, scp1, scp2, ssq):
    r = pl.program_id(0)
    rc = rc_ref[0]
    hc = hc_ref[0]

    def wmat(c, b_ref, l):
        return c[l:l + 1, 0:1] * b_ref[l, 0] + c[l:l + 1, 1:2] * b_ref[l, 1]

    bf = jnp.bfloat16
    a = a_ref[0].astype(bf)
    a2 = a2_ref[0].astype(bf)
    x1 = s1_ref[0]
    x2 = s2_ref[0]

    def layer(adjb, u, v, w):
        xw = jnp.concatenate([_dotb(u, w), _dotb(v, w)], axis=1).astype(bf)
        y = jax.lax.dot_general(adjb, xw, (((1,), (0,)), ((), ())),
                                preferred_element_type=jnp.float32)
        y = jnp.maximum(y, 0.0)
        return y[:, :D], y[:, D:]

    p1, q1 = layer(a, x1, x2, wmat(rc, rb_ref, 0))
    p2, q2 = layer(a2, x1, x2, wmat(hc, hb_ref, 0))
    p1, q1 = layer(a, p1, q1, wmat(rc, rb_ref, 1))
    p2, q2 = layer(a2, p2, q2, wmat(hc, hb_ref, 1))

    cp1 = jax.nn.sigmoid(jnp.mean(p1, axis=0, keepdims=True))
    cp2 = jax.nn.sigmoid(jnp.mean(p2, axis=0, keepdims=True))

    w1 = w1_ref[...]
    w2 = w2_ref[...]

    def seg(c, h, w):
        return _dot_tb(_dot_tb(c, w), h)

    def disc_block(c1, c2, hb1, hb2, he1, he2):
        return jnp.concatenate(
            [seg(c1, hb2, w1), seg(c2, hb1, w2),
             seg(c1, he2, w1), seg(c2, he1, w2)], axis=0)[None]

    logits_ref[pl.ds(1 + r, 1)] = disc_block(cp1, cp2, p1, p2, q1, q2)

    hp_i = (p1 + p2) * 0.5
    hn_i = (q1 + q2) * 0.5
    sq_r = _sum11(hp_i * hp_i - hn_i * hn_i)

    @pl.when(r == 0)
    def _():
        shp1[...] = p1
        shp2[...] = p2
        shn1[...] = q1
        shn2[...] = q2
        scp1[...] = cp1
        scp2[...] = cp2
        ssq[...] = sq_r

    @pl.when(r > 0)
    def _():
        shp1[...] += p1
        shp2[...] += p2
        shn1[...] += q1
        shn2[...] += q2
        scp1[...] += cp1
        scp2[...] += cp2
        ssq[...] += sq_r

    @pl.when(r == R - 1)
    def _():
        inv = 1.0 / R
        hp1_all = shp1[...] * inv
        hp2_all = shp2[...] * inv
        hn1_all = shn1[...] * inv
        hn2_all = shn2[...] * inv
        c1_all = scp1[...] * inv
        c2_all = scp2[...] * inv

        logits_ref[pl.ds(0, 1)] = disc_block(c1_all, c2_all, hp1_all, hp2_all,
                                             hn1_all, hn2_all)

        amat = (hp1_all + hp2_all) * 0.5
        aneg = (hn1_all + hn2_all) * 0.5
        reg = ssq[...] - 2.0 * R * _sum11(amat * (amat - aneg))

        def proj(h):
            z = _dot_tb(h, f1w_ref[...]) + f1b_ref[...]
            z = jnp.where(z > 0.0, z, jnp.exp(jnp.minimum(z, 0.0)) - 1.0)
            return _dot_tb(z, f2w_ref[...]) + f2b_ref[...]

        z1 = proj(hp1_all)
        z2 = proj(hp2_all)
        zk = _dotb(z1, fkw_ref[...])
        fkb = fkb_ref[0:1, 0:1]

        NR = 576
        zks = zk[:NR]
        m1t = _dot_tb(z1, zks)
        m2t = _dot_tb(z2, zks)

        rowj = jax.lax.broadcasted_iota(jnp.int32, (N, NR), 0)
        coli = jax.lax.broadcasted_iota(jnp.int32, (N, NR), 1)
        mask7 = rowj == ((7 * coli + 1) & (N - 1))
        mask13 = rowj == ((13 * coli + 5) & (N - 1))

        def diag(m, mask):
            return jnp.sum(jnp.where(mask, m, 0.0), axis=0, keepdims=True) + fkb

        d7_1 = diag(m1t, mask7)
        d7_2 = diag(m2t, mask7)
        d13_1 = diag(m1t, mask13)
        d13_2 = diag(m2t, mask13)

        def bce_pos(v):
            return _sum11(jnp.maximum(v, 0.0) - v
                          + jnp.log1p(jnp.exp(-jnp.abs(v))))

        def bce_neg(v):
            return _sum11(jnp.maximum(v, 0.0) + jnp.log1p(jnp.exp(-jnp.abs(v))))

        node = jnp.zeros((1, 1), jnp.float32)
        for i in range(R):
            a0 = 17 * i
            node = node + (bce_pos(d7_1[:, a0:a0 + S])
                           + bce_pos(d7_2[:, a0:a0 + S])
                           + bce_neg(d13_1[:, a0:a0 + S])
                           + bce_neg(d13_2[:, a0:a0 + S])) * (1.0 / (4 * S))

        lane = jax.lax.broadcasted_iota(jnp.int32, (8, 128), 1)
        sub = jax.lax.broadcasted_iota(jnp.int32, (8, 128), 0)
        regb = jnp.broadcast_to(reg, (8, 128))
        nodeb = jnp.broadcast_to(node, (8, 128))
        misc_ref[...] = jnp.where((sub == 0) & (lane == 0), regb,
                                  jnp.where((sub == 0) & (lane == 1), nodeb,
                                            0.0))


def kernel(seq1, seq2, adj, adj_2, sparse, rgcn_bases, rgcn_comp, hrgcn_bases,
           hrgcn_comp, disc_W1, disc_W2, fc1_w, fc1_b, fc2_w, fc2_b, fk_w, fk_b):
    rc_t = jnp.transpose(rgcn_comp, (1, 0, 2))
    hc_t = jnp.transpose(hrgcn_comp, (1, 0, 2))

    full = lambda shape: pl.BlockSpec(shape, lambda r: (0,) * len(shape))
    per_r3 = lambda d1, d2: pl.BlockSpec((1, d1, d2), lambda r: (r, 0, 0))

    logits, misc = pl.pallas_call(
        _fused_kernel,
        grid=(R,),
        in_specs=[
            per_r3(L, B), per_r3(L, B),
            full((L, B, D, D)), full((L, B, D, D)),
            per_r3(N, D), per_r3(N, D),
            per_r3(N, N), per_r3(N, N),
            full((D, D)), full((D, D)),
            full((D, D)), full((1, D)), full((D, D)), full((1, D)),
            full((D, D)), full((1, 1)),
        ],
        out_specs=[full((4, 4, N)), full((8, 128))],
        out_shape=[jax.ShapeDtypeStruct((4, 4, N), jnp.float32),
                   jax.ShapeDtypeStruct((8, 128), jnp.float32)],
        scratch_shapes=[pltpu.VMEM((N, D), jnp.float32)] * 4
        + [pltpu.VMEM((1, D), jnp.float32)] * 2
        + [pltpu.VMEM((1, 1), jnp.float32)],
        interpret=_INTERPRET,
    )(rc_t, hc_t, rgcn_bases, hrgcn_bases, seq1, seq2, adj, adj_2,
      disc_W1, disc_W2, fc1_w, fc1_b.reshape(1, D), fc2_w, fc2_b.reshape(1, D),
      fk_w, fk_b.reshape(1, 1))

    return jnp.concatenate([logits.reshape(-1), misc[0, :2]])


# speedup vs baseline: 1.0190x; 1.0083x over previous
import jax
import jax.numpy as jnp
from jax.experimental import pallas as pl
from jax.experimental.pallas import tpu as pltpu

R, N, D, B, L, S = 3, 1024, 256, 2, 2, 512

_INTERPRET = False


def _dot(x, w):
    return jax.lax.dot_general(x, w, (((1,), (0,)), ((), ())),
                               preferred_element_type=jnp.float32)


def _dotb(x, w):
    return jax.lax.dot_general(x.astype(jnp.bfloat16), w.astype(jnp.bfloat16),
                               (((1,), (0,)), ((), ())),
                               preferred_element_type=jnp.float32)


def _dot_tb(x, w):
    return jax.lax.dot_general(x.astype(jnp.bfloat16), w.astype(jnp.bfloat16),
                               (((1,), (1,)), ((), ())),
                               preferred_element_type=jnp.float32)


def _dot_t(x, w):
    return jax.lax.dot_general(x, w, (((1,), (1,)), ((), ())),
                               preferred_element_type=jnp.float32)


def _sum11(x):
    return jnp.sum(x, axis=1, keepdims=True).sum(axis=0, keepdims=True)


def _fused_kernel(rc_ref, hc_ref, rb_ref, hb_ref, s1_ref, s2_ref, a_ref, a2_ref,
                  w1_ref, w2_ref, f1w_ref, f1b_ref, f2w_ref, f2b_ref,
                  fkw_ref, fkb_ref,
                  out_ref,
                  shp1, shp2, shn1, shn2, scp1, scp2, ssq):
    r = pl.program_id(0)
    rc = rc_ref[:, pl.ds(r, 1), :][:, 0, :]
    hc = hc_ref[:, pl.ds(r, 1), :][:, 0, :]

    def wmat(c, b_ref, l):
        return c[l:l + 1, 0:1] * b_ref[l, 0] + c[l:l + 1, 1:2] * b_ref[l, 1]

    bf = jnp.bfloat16
    a = a_ref[0].astype(bf)
    a2 = a2_ref[0].astype(bf)
    x1 = s1_ref[0]
    x2 = s2_ref[0]

    def layer(adjb, u, v, w):
        xw = jnp.concatenate([_dotb(u, w), _dotb(v, w)], axis=1).astype(bf)
        y = jax.lax.dot_general(adjb, xw, (((1,), (0,)), ((), ())),
                                preferred_element_type=jnp.float32)
        y = jnp.maximum(y, 0.0)
        return y[:, :D], y[:, D:]

    p1, q1 = layer(a, x1, x2, wmat(rc, rb_ref, 0))
    p2, q2 = layer(a2, x1, x2, wmat(hc, hb_ref, 0))
    p1, q1 = layer(a, p1, q1, wmat(rc, rb_ref, 1))
    p2, q2 = layer(a2, p2, q2, wmat(hc, hb_ref, 1))

    cp1 = jax.nn.sigmoid(jnp.mean(p1, axis=0, keepdims=True))
    cp2 = jax.nn.sigmoid(jnp.mean(p2, axis=0, keepdims=True))

    w1 = w1_ref[...]
    w2 = w2_ref[...]

    def seg(c, h, w):
        return _dot_tb(_dot_tb(c, w), h)

    def disc_block(d_idx, c1, c2, hb1, hb2, he1, he2):
        row = jnp.concatenate(
            [seg(c1, hb2, w1), seg(c2, hb1, w2),
             seg(c1, he2, w1), seg(c2, he1, w2)], axis=1)
        out_ref[pl.ds(d_idx * 4 * N, 4 * N)] = row.reshape(4 * N)

    disc_block(1 + r, cp1, cp2, p1, p2, q1, q2)

    hp_i = (p1 + p2) * 0.5
    hn_i = (q1 + q2) * 0.5
    sq_r = _sum11(hp_i * hp_i - hn_i * hn_i)

    @pl.when(r == 0)
    def _():
        shp1[...] = p1
        shp2[...] = p2
        shn1[...] = q1
        shn2[...] = q2
        scp1[...] = cp1
        scp2[...] = cp2
        ssq[...] = sq_r

    @pl.when(r > 0)
    def _():
        shp1[...] += p1
        shp2[...] += p2
        shn1[...] += q1
        shn2[...] += q2
        scp1[...] += cp1
        scp2[...] += cp2
        ssq[...] += sq_r

    @pl.when(r == R - 1)
    def _():
        inv = 1.0 / R
        hp1_all = shp1[...] * inv
        hp2_all = shp2[...] * inv
        hn1_all = shn1[...] * inv
        hn2_all = shn2[...] * inv
        c1_all = scp1[...] * inv
        c2_all = scp2[...] * inv

        disc_block(0, c1_all, c2_all, hp1_all, hp2_all, hn1_all, hn2_all)

        amat = (hp1_all + hp2_all) * 0.5
        aneg = (hn1_all + hn2_all) * 0.5
        reg = ssq[...] - 2.0 * R * _sum11(amat * (amat - aneg))

        def proj(h):
            z = _dot_tb(h, f1w_ref[...]) + f1b_ref[...]
            z = jnp.where(z > 0.0, z, jnp.exp(jnp.minimum(z, 0.0)) - 1.0)
            return _dot_tb(z, f2w_ref[...]) + f2b_ref[...]

        z1 = proj(hp1_all)
        z2 = proj(hp2_all)
        zk = _dotb(z1, fkw_ref[...])
        fkb = fkb_ref[0:1, 0:1]

        NR = 576
        zks = zk[:NR]
        m1t = _dot_tb(z1, zks)
        m2t = _dot_tb(z2, zks)

        rowj = jax.lax.broadcasted_iota(jnp.int32, (N, NR), 0)
        coli = jax.lax.broadcasted_iota(jnp.int32, (N, NR), 1)
        mask7 = rowj == ((7 * coli + 1) & (N - 1))
        mask13 = rowj == ((13 * coli + 5) & (N - 1))

        def diag(m, mask):
            return jnp.sum(jnp.where(mask, m, 0.0), axis=0, keepdims=True) + fkb

        d7_1 = diag(m1t, mask7)
        d7_2 = diag(m2t, mask7)
        d13_1 = diag(m1t, mask13)
        d13_2 = diag(m2t, mask13)

        def bce_pos(v):
            return _sum11(jnp.maximum(v, 0.0) - v
                          + jnp.log1p(jnp.exp(-jnp.abs(v))))

        def bce_neg(v):
            return _sum11(jnp.maximum(v, 0.0) + jnp.log1p(jnp.exp(-jnp.abs(v))))

        node = jnp.zeros((1, 1), jnp.float32)
        for i in range(R):
            a0 = 17 * i
            node = node + (bce_pos(d7_1[:, a0:a0 + S])
                           + bce_pos(d7_2[:, a0:a0 + S])
                           + bce_neg(d13_1[:, a0:a0 + S])
                           + bce_neg(d13_2[:, a0:a0 + S])) * (1.0 / (4 * S))

        pair = jnp.concatenate([reg, node], axis=1).reshape(2)
        out_ref[pl.ds(16 * N, 2)] = pair


def kernel(seq1, seq2, adj, adj_2, sparse, rgcn_bases, rgcn_comp, hrgcn_bases,
           hrgcn_comp, disc_W1, disc_W2, fc1_w, fc1_b, fc2_w, fc2_b, fk_w, fk_b):
    full = lambda shape: pl.BlockSpec(shape, lambda r: (0,) * len(shape))
    per_r3 = lambda d1, d2: pl.BlockSpec((1, d1, d2), lambda r: (r, 0, 0))

    out = pl.pallas_call(
        _fused_kernel,
        grid=(R,),
        in_specs=[
            full((L, R, B)), full((L, R, B)),
            full((L, B, D, D)), full((L, B, D, D)),
            per_r3(N, D), per_r3(N, D),
            per_r3(N, N), per_r3(N, N),
            full((D, D)), full((D, D)),
            full((D, D)), full((1, D)), full((D, D)), full((1, D)),
            full((D, D)), full((1, 1)),
        ],
        out_specs=pl.BlockSpec((16 * N + 2,), lambda r: (0,)),
        out_shape=jax.ShapeDtypeStruct((16 * N + 2,), jnp.float32),
        scratch_shapes=[pltpu.VMEM((N, D), jnp.float32)] * 4
        + [pltpu.VMEM((1, D), jnp.float32)] * 2
        + [pltpu.VMEM((1, 1), jnp.float32)],
        interpret=_INTERPRET,
    )(rgcn_comp, hrgcn_comp, rgcn_bases, hrgcn_bases, seq1, seq2, adj, adj_2,
      disc_W1, disc_W2, fc1_w, fc1_b.reshape(1, D), fc2_w, fc2_b.reshape(1, D),
      fk_w, fk_b.reshape(1, 1))

    return out
